# Initial kernel scaffold; baseline (speedup 1.0000x reference)
import jax
import jax.numpy as jnp
from jax.experimental import pallas as pl


def kernel(x, edge_index, Wp, a_src, a_tgt, Wskip):
    raise NotImplementedError("write your pallas kernel here")


# speedup vs baseline: 34.5061x; 34.5061x over previous
import functools

import jax
import jax.numpy as jnp
from jax import lax
from jax.experimental import pallas as pl
from jax.experimental.pallas import tpu as pltpu
from jax.experimental.pallas import tpu_sc as plsc

H = 4
F = 32
HF = H * F
NC = 2
NS = 16
NW = NC * NS
C = 80


def _leaky(x):
    return jnp.where(x >= 0.0, x, 0.2 * x)


def _stage_a_body(nblk, x_ref, wp_ref, asrc_ref, atgt_ref, wskip_ref,
                  proj_ref, ss_ref, st_ref, skip_ref, m_ref, mx_ref):
    i = pl.program_id(0)
    xb = x_ref[...]
    dot = functools.partial(
        lax.dot_general,
        dimension_numbers=(((1,), (0,)), ((), ())),
        preferred_element_type=jnp.float32,
        precision=lax.Precision.HIGHEST,
    )
    proj = dot(xb, wp_ref[...])
    proj_ref[...] = proj
    ss = dot(proj, asrc_ref[...])
    ss_ref[...] = ss
    st = dot(proj, atgt_ref[...])
    st_ref[...] = st
    skip_ref[...] = dot(xb, wskip_ref[...])
    m1 = jnp.max(ss)
    m2 = jnp.max(st)

    @pl.when(i == 0)
    def _():
        mx_ref[0] = m1
        mx_ref[1] = m2

    @pl.when(i > 0)
    def _():
        mx_ref[0] = jnp.maximum(mx_ref[0], m1)
        mx_ref[1] = jnp.maximum(mx_ref[1], m2)

    @pl.when(i == nblk - 1)
    def _():
        m_ref[...] = jnp.full((8, 128), _leaky(mx_ref[0] + mx_ref[1]),
                              jnp.float32)


def _stage_a(x, Wp, Asrc16, Atgt16, Wskip, N):
    R = 1000
    nblk = N // R
    return pl.pallas_call(
        functools.partial(_stage_a_body, nblk),
        grid=(nblk,),
        in_specs=[
            pl.BlockSpec((R, HF), lambda i: (i, 0)),
            pl.BlockSpec((HF, HF), lambda i: (0, 0)),
            pl.BlockSpec((HF, 16), lambda i: (0, 0)),
            pl.BlockSpec((HF, 16), lambda i: (0, 0)),
            pl.BlockSpec((HF, HF), lambda i: (0, 0)),
        ],
        out_specs=[
            pl.BlockSpec((R, HF), lambda i: (i, 0)),
            pl.BlockSpec((R, 16), lambda i: (i, 0)),
            pl.BlockSpec((R, 16), lambda i: (i, 0)),
            pl.BlockSpec((R, HF), lambda i: (i, 0)),
            pl.BlockSpec((8, 128), lambda i: (0, 0)),
        ],
        out_shape=[
            jax.ShapeDtypeStruct((N, HF), jnp.float32),
            jax.ShapeDtypeStruct((N, 16), jnp.float32),
            jax.ShapeDtypeStruct((N, 16), jnp.float32),
            jax.ShapeDtypeStruct((N, HF), jnp.float32),
            jax.ShapeDtypeStruct((8, 128), jnp.float32),
        ],
        scratch_shapes=[pltpu.SMEM((2,), jnp.float32)],
    )(x, Wp, Asrc16, Atgt16, Wskip)


def _edge_pass(proj, ssrc, stgt, esrc, etgt, m16, zerosA, zerosD, N, NP, E):
    EW = E // NW
    NCH = EW // C
    NZR = NP // NS
    mesh = plsc.VectorSubcoreMesh(core_axis_name="c", subcore_axis_name="s",
                                  num_cores=NC, num_subcores=NS)

    @functools.partial(
        pl.kernel,
        out_type=(
            jax.ShapeDtypeStruct((NC, NP, HF), jnp.float32),
            jax.ShapeDtypeStruct((NC, NP, 16), jnp.float32),
        ),
        mesh=mesh,
        scratch_types=[
            pltpu.VMEM_SHARED((NP, HF), jnp.float32),
            pltpu.VMEM_SHARED((NP, 16), jnp.float32),
            pltpu.VMEM((C,), jnp.int32),
            pltpu.VMEM((C,), jnp.int32),
            pltpu.VMEM((C, HF), jnp.float32),
            pltpu.VMEM((C, 16), jnp.float32),
            pltpu.VMEM((C, 16), jnp.float32),
            pltpu.VMEM((C, 16), jnp.float32),
            pltpu.VMEM((16,), jnp.float32),
        ],
        compiler_params=pltpu.CompilerParams(use_tc_tiling_on_sc=False,
                                             needs_layout_passes=False),
    )
    def k(proj_hbm, ssrc_hbm, stgt_hbm, esrc_hbm, etgt_hbm, m_hbm,
          zerosA_hbm, zerosD_hbm, acc_out, den_out,
          acc_sh, den_sh, src_v, tgt_v, pbuf, wbuf, ssb, stb, m_v):
        cid = lax.axis_index("c")
        sid = lax.axis_index("s")
        wid = cid * NS + sid
        r0 = sid * NZR
        pltpu.sync_copy(zerosA_hbm, pbuf)
        pltpu.sync_copy(zerosD_hbm, wbuf)

        def zero_j(j, carry):
            pltpu.sync_copy(pbuf, acc_sh.at[pl.ds(r0 + j * C, C)])
            pltpu.sync_copy(wbuf, den_sh.at[pl.ds(r0 + j * C, C)])
            return carry

        lax.fori_loop(0, NZR // C, zero_j, 0)
        pltpu.sync_copy(m_hbm, m_v)
        m = m_v[...]
        lanes = lax.iota(jnp.int32, 16)
        maskH = lanes < H
        plsc.subcore_barrier()

        def chunk(kk, carry):
            off = (wid * NCH + kk) * C
            pltpu.sync_copy(esrc_hbm.at[pl.ds(off, C)], src_v)
            pltpu.sync_copy(etgt_hbm.at[pl.ds(off, C)], tgt_v)
            pltpu.sync_copy(proj_hbm.at[src_v], pbuf)
            pltpu.sync_copy(ssrc_hbm.at[src_v], ssb)
            pltpu.sync_copy(stgt_hbm.at[tgt_v], stb)

            def edge(e, carry2):
                ev = jnp.full((16,), e, jnp.int32)
                srow = plsc.load_gather(ssb, [ev, lanes])
                trow = plsc.load_gather(stb, [ev, lanes])
                z = srow + trow
                z = jnp.where(z >= 0.0, z, 0.2 * z)
                w = jnp.where(maskH, jnp.exp(z - m), 0.0)
                plsc.store_scatter(wbuf, [ev, lanes], w)
                for h in range(H):
                    hv = jnp.full((16,), h, jnp.int32)
                    b = plsc.load_gather(wbuf, [ev, hv])
                    for t in range(2):
                        cv = lanes + (2 * h + t) * 16
                        v = plsc.load_gather(pbuf, [ev, cv]) * b
                        plsc.store_scatter(pbuf, [ev, cv], v)
                return carry2

            lax.fori_loop(0, C, edge, 0)
            pltpu.sync_copy(pbuf, acc_sh.at[tgt_v], add=True)
            pltpu.sync_copy(wbuf, den_sh.at[tgt_v], add=True)
            return carry

        lax.fori_loop(0, NCH, chunk, 0)
        plsc.subcore_barrier()

        def dump_j(j, carry):
            rj = r0 + j * C
            pltpu.sync_copy(acc_sh.at[pl.ds(rj, C)], pbuf)
            pltpu.sync_copy(pbuf, acc_out.at[cid, pl.ds(rj, C)])
            pltpu.sync_copy(den_sh.at[pl.ds(rj, C)], wbuf)
            pltpu.sync_copy(wbuf, den_out.at[cid, pl.ds(rj, C)])
            return carry

        lax.fori_loop(0, NZR // C, dump_j, 0)

    return k(proj, ssrc, stgt, esrc, etgt, m16, zerosA, zerosD)


def _stage_c_body(acc_ref, den_ref, skip_ref, g2_ref, out_ref):
    a = acc_ref[0] + acc_ref[1]
    d = den_ref[0] + den_ref[1]
    d128 = lax.dot_general(d, g2_ref[...], (((1,), (0,)), ((), ())),
                           preferred_element_type=jnp.float32,
                           precision=lax.Precision.HIGHEST)
    out_ref[...] = a / (d128 + 1e-16) + skip_ref[...]


def _stage_c(accP, denP, skip, G2, N):
    R = 1000
    nblk = N // R
    return pl.pallas_call(
        _stage_c_body,
        grid=(nblk,),
        in_specs=[
            pl.BlockSpec((NC, R, HF), lambda i: (0, i, 0)),
            pl.BlockSpec((NC, R, 16), lambda i: (0, i, 0)),
            pl.BlockSpec((R, HF), lambda i: (i, 0)),
            pl.BlockSpec((16, HF), lambda i: (0, 0)),
        ],
        out_specs=pl.BlockSpec((R, HF), lambda i: (i, 0)),
        out_shape=jax.ShapeDtypeStruct((N, HF), jnp.float32),
    )(accP, denP, skip, G2)


def kernel(x, edge_index, Wp, a_src, a_tgt, Wskip):
    N = x.shape[0]
    E = edge_index.shape[1]
    eyeH = jnp.eye(H, dtype=jnp.float32)
    A_src = (a_src[0][:, :, None] * eyeH[:, None, :]).reshape(HF, H)
    A_tgt = (a_tgt[0][:, :, None] * eyeH[:, None, :]).reshape(HF, H)
    pad = jnp.zeros((HF, 16 - H), jnp.float32)
    Asrc16 = jnp.concatenate([A_src, pad], axis=1)
    Atgt16 = jnp.concatenate([A_tgt, pad], axis=1)
    G2 = jnp.concatenate(
        [jnp.kron(eyeH, jnp.ones((1, F), jnp.float32)),
         jnp.zeros((16 - H, HF), jnp.float32)], axis=0)
    esrc = edge_index[0].astype(jnp.int32)
    etgt = edge_index[1].astype(jnp.int32)
    NP = ((N + NS * C - 1) // (NS * C)) * NS * C
    zerosA = jnp.zeros((C, HF), jnp.float32)
    zerosD = jnp.zeros((C, 16), jnp.float32)

    proj, ssrc, stgt, skip, m8 = _stage_a(x, Wp, Asrc16, Atgt16, Wskip, N)
    m16 = m8.reshape(-1)[:16]
    accP, denP = _edge_pass(proj, ssrc, stgt, esrc, etgt, m16,
                            zerosA, zerosD, N, NP, E)
    return _stage_c(accP, denP, skip, G2, N)


# speedup vs baseline: 41.2871x; 1.1965x over previous
import functools

import jax
import jax.numpy as jnp
from jax import lax
from jax.experimental import pallas as pl
from jax.experimental.pallas import tpu as pltpu
from jax.experimental.pallas import tpu_sc as plsc

H = 4
F = 32
HF = H * F
NC = 2
NS = 16
NW = NC * NS
C = 80


def _leaky(x):
    return jnp.where(x >= 0.0, x, 0.2 * x)


def _stage_a_body(nblk, x_ref, wp_ref, asrc_ref, atgt_ref, wskip_ref,
                  proj_ref, ss_ref, st_ref, skip_ref, m_ref, mx_ref):
    i = pl.program_id(0)
    xb = x_ref[...]
    dot = functools.partial(
        lax.dot_general,
        dimension_numbers=(((1,), (0,)), ((), ())),
        preferred_element_type=jnp.float32,
        precision=lax.Precision.HIGHEST,
    )
    proj = dot(xb, wp_ref[...])
    proj_ref[...] = proj
    ss = dot(proj, asrc_ref[...])
    ss_ref[...] = ss
    st = dot(proj, atgt_ref[...])
    st_ref[...] = st
    skip_ref[...] = dot(xb, wskip_ref[...])
    m1 = jnp.max(ss)
    m2 = jnp.max(st)

    @pl.when(i == 0)
    def _():
        mx_ref[0] = m1
        mx_ref[1] = m2

    @pl.when(i > 0)
    def _():
        mx_ref[0] = jnp.maximum(mx_ref[0], m1)
        mx_ref[1] = jnp.maximum(mx_ref[1], m2)

    @pl.when(i == nblk - 1)
    def _():
        m_ref[...] = jnp.full((8, 128), _leaky(mx_ref[0] + mx_ref[1]),
                              jnp.float32)


def _stage_a(x, Wp, Asrc16, Atgt16, Wskip, N):
    R = 1000
    nblk = N // R
    return pl.pallas_call(
        functools.partial(_stage_a_body, nblk),
        grid=(nblk,),
        in_specs=[
            pl.BlockSpec((R, HF), lambda i: (i, 0)),
            pl.BlockSpec((HF, HF), lambda i: (0, 0)),
            pl.BlockSpec((HF, 16), lambda i: (0, 0)),
            pl.BlockSpec((HF, 16), lambda i: (0, 0)),
            pl.BlockSpec((HF, HF), lambda i: (0, 0)),
        ],
        out_specs=[
            pl.BlockSpec((R, HF), lambda i: (i, 0)),
            pl.BlockSpec((R, 16), lambda i: (i, 0)),
            pl.BlockSpec((R, 16), lambda i: (i, 0)),
            pl.BlockSpec((R, HF), lambda i: (i, 0)),
            pl.BlockSpec((8, 128), lambda i: (0, 0)),
        ],
        out_shape=[
            jax.ShapeDtypeStruct((N, HF), jnp.float32),
            jax.ShapeDtypeStruct((N, 16), jnp.float32),
            jax.ShapeDtypeStruct((N, 16), jnp.float32),
            jax.ShapeDtypeStruct((N, HF), jnp.float32),
            jax.ShapeDtypeStruct((8, 128), jnp.float32),
        ],
        scratch_shapes=[pltpu.SMEM((2,), jnp.float32)],
    )(x, Wp, Asrc16, Atgt16, Wskip)


def _edge_pass(proj, ssrc, stgt, esrc, etgt, m16, zerosA, zerosD, N, NP, E):
    EW = E // NW
    NCH = EW // C
    NZR = NP // NS
    mesh = plsc.VectorSubcoreMesh(core_axis_name="c", subcore_axis_name="s",
                                  num_cores=NC, num_subcores=NS)

    @functools.partial(
        pl.kernel,
        out_type=(
            jax.ShapeDtypeStruct((NC, NP, HF), jnp.float32),
            jax.ShapeDtypeStruct((NC, NP, 16), jnp.float32),
        ),
        mesh=mesh,
        scratch_types=[
            pltpu.VMEM_SHARED((NP, HF), jnp.float32),
            pltpu.VMEM_SHARED((NP, 16), jnp.float32),
            pltpu.VMEM((C,), jnp.int32),
            pltpu.VMEM((C,), jnp.int32),
            pltpu.VMEM((C, HF), jnp.float32),
            pltpu.VMEM((C, 16), jnp.float32),
            pltpu.VMEM((C, 16), jnp.float32),
            pltpu.VMEM((C, 16), jnp.float32),
            pltpu.VMEM((16,), jnp.float32),
            pltpu.SemaphoreType.DMA,
        ],
        compiler_params=pltpu.CompilerParams(use_tc_tiling_on_sc=False,
                                             needs_layout_passes=False),
    )
    def k(proj_hbm, ssrc_hbm, stgt_hbm, esrc_hbm, etgt_hbm, m_hbm,
          zerosA_hbm, zerosD_hbm, acc_out, den_out,
          acc_sh, den_sh, src_v, tgt_v, pbuf, wbuf, ssb, stb, m_v, sem):
        cid = lax.axis_index("c")
        sid = lax.axis_index("s")
        wid = cid * NS + sid
        r0 = sid * NZR
        pltpu.sync_copy(zerosA_hbm, pbuf)
        pltpu.sync_copy(zerosD_hbm, wbuf)

        def zero_j(j, carry):
            pltpu.sync_copy(pbuf, acc_sh.at[pl.ds(r0 + j * C, C)])
            pltpu.sync_copy(wbuf, den_sh.at[pl.ds(r0 + j * C, C)])
            return carry

        lax.fori_loop(0, NZR // C, zero_j, 0)
        pltpu.sync_copy(m_hbm, m_v)
        m = m_v[...]
        lanes = lax.iota(jnp.int32, 16)
        maskH = lanes < H
        plsc.subcore_barrier()

        def chunk(kk, carry):
            off = (wid * NCH + kk) * C
            c1 = pltpu.async_copy(esrc_hbm.at[pl.ds(off, C)], src_v, sem)
            c2 = pltpu.async_copy(etgt_hbm.at[pl.ds(off, C)], tgt_v, sem)
            c1.wait()
            c2.wait()
            g1 = pltpu.async_copy(proj_hbm.at[src_v], pbuf, sem)
            g2 = pltpu.async_copy(ssrc_hbm.at[src_v], ssb, sem)
            g3 = pltpu.async_copy(stgt_hbm.at[tgt_v], stb, sem)
            g1.wait()
            g2.wait()
            g3.wait()

            def edge(e, carry2):
                ev = jnp.full((16,), e, jnp.int32)
                srow = plsc.load_gather(ssb, [ev, lanes])
                trow = plsc.load_gather(stb, [ev, lanes])
                z = srow + trow
                z = jnp.where(z >= 0.0, z, 0.2 * z)
                w = jnp.where(maskH, jnp.exp(z - m), 0.0)
                plsc.store_scatter(wbuf, [ev, lanes], w)
                for h in range(H):
                    hv = jnp.full((16,), h, jnp.int32)
                    b = plsc.load_gather(wbuf, [ev, hv])
                    for t in range(2):
                        cv = lanes + (2 * h + t) * 16
                        v = plsc.load_gather(pbuf, [ev, cv]) * b
                        plsc.store_scatter(pbuf, [ev, cv], v)
                return carry2

            lax.fori_loop(0, C, edge, 0)
            pltpu.sync_copy(pbuf, acc_sh.at[tgt_v], add=True)
            pltpu.sync_copy(wbuf, den_sh.at[tgt_v], add=True)
            return carry

        lax.fori_loop(0, NCH, chunk, 0)
        plsc.subcore_barrier()

        def dump_j(j, carry):
            rj = r0 + j * C
            pltpu.sync_copy(acc_sh.at[pl.ds(rj, C)], pbuf)
            pltpu.sync_copy(pbuf, acc_out.at[cid, pl.ds(rj, C)])
            pltpu.sync_copy(den_sh.at[pl.ds(rj, C)], wbuf)
            pltpu.sync_copy(wbuf, den_out.at[cid, pl.ds(rj, C)])
            return carry

        lax.fori_loop(0, NZR // C, dump_j, 0)

    return k(proj, ssrc, stgt, esrc, etgt, m16, zerosA, zerosD)


def _stage_c_body(acc_ref, den_ref, skip_ref, g2_ref, out_ref):
    a = acc_ref[0] + acc_ref[1]
    d = den_ref[0] + den_ref[1]
    d128 = lax.dot_general(d, g2_ref[...], (((1,), (0,)), ((), ())),
                           preferred_element_type=jnp.float32,
                           precision=lax.Precision.HIGHEST)
    out_ref[...] = a / (d128 + 1e-16) + skip_ref[...]


def _stage_c(accP, denP, skip, G2, N):
    R = 1000
    nblk = N // R
    return pl.pallas_call(
        _stage_c_body,
        grid=(nblk,),
        in_specs=[
            pl.BlockSpec((NC, R, HF), lambda i: (0, i, 0)),
            pl.BlockSpec((NC, R, 16), lambda i: (0, i, 0)),
            pl.BlockSpec((R, HF), lambda i: (i, 0)),
            pl.BlockSpec((16, HF), lambda i: (0, 0)),
        ],
        out_specs=pl.BlockSpec((R, HF), lambda i: (i, 0)),
        out_shape=jax.ShapeDtypeStruct((N, HF), jnp.float32),
    )(accP, denP, skip, G2)


def kernel(x, edge_index, Wp, a_src, a_tgt, Wskip):
    N = x.shape[0]
    E = edge_index.shape[1]
    eyeH = jnp.eye(H, dtype=jnp.float32)
    A_src = (a_src[0][:, :, None] * eyeH[:, None, :]).reshape(HF, H)
    A_tgt = (a_tgt[0][:, :, None] * eyeH[:, None, :]).reshape(HF, H)
    pad = jnp.zeros((HF, 16 - H), jnp.float32)
    Asrc16 = jnp.concatenate([A_src, pad], axis=1)
    Atgt16 = jnp.concatenate([A_tgt, pad], axis=1)
    G2 = jnp.concatenate(
        [jnp.kron(eyeH, jnp.ones((1, F), jnp.float32)),
         jnp.zeros((16 - H, HF), jnp.float32)], axis=0)
    esrc = edge_index[0].astype(jnp.int32)
    etgt = edge_index[1].astype(jnp.int32)
    NP = ((N + NS * C - 1) // (NS * C)) * NS * C
    zerosA = jnp.zeros((C, HF), jnp.float32)
    zerosD = jnp.zeros((C, 16), jnp.float32)

    proj, ssrc, stgt, skip, m8 = _stage_a(x, Wp, Asrc16, Atgt16, Wskip, N)
    m16 = m8.reshape(-1)[:16]
    accP, denP = _edge_pass(proj, ssrc, stgt, esrc, etgt, m16,
                            zerosA, zerosD, N, NP, E)
    return _stage_c(accP, denP, skip, G2, N)


# speedup vs baseline: 45.7650x; 1.1085x over previous
import functools

import jax
import jax.numpy as jnp
from jax import lax
from jax.experimental import pallas as pl
from jax.experimental.pallas import tpu as pltpu
from jax.experimental.pallas import tpu_sc as plsc

H = 4
F = 32
HF = H * F
NC = 2
NS = 16
NW = NC * NS
C = 80


def _leaky(x):
    return jnp.where(x >= 0.0, x, 0.2 * x)


def _stage_a_body(nblk, x_ref, wp_ref, asrc_ref, atgt_ref, wskip_ref,
                  proj_ref, ss_ref, st_ref, skip_ref, m_ref, mx_ref):
    i = pl.program_id(0)
    xb = x_ref[...]
    dot = functools.partial(
        lax.dot_general,
        dimension_numbers=(((1,), (0,)), ((), ())),
        preferred_element_type=jnp.float32,
        precision=lax.Precision.HIGHEST,
    )
    proj = dot(xb, wp_ref[...])
    proj_ref[...] = proj
    ss = dot(proj, asrc_ref[...])
    ss_ref[...] = ss
    st = dot(proj, atgt_ref[...])
    st_ref[...] = st
    skip_ref[...] = dot(xb, wskip_ref[...])
    m1 = jnp.max(ss)
    m2 = jnp.max(st)

    @pl.when(i == 0)
    def _():
        mx_ref[0] = m1
        mx_ref[1] = m2

    @pl.when(i > 0)
    def _():
        mx_ref[0] = jnp.maximum(mx_ref[0], m1)
        mx_ref[1] = jnp.maximum(mx_ref[1], m2)

    @pl.when(i == nblk - 1)
    def _():
        m_ref[...] = jnp.full((8, 128), _leaky(mx_ref[0] + mx_ref[1]),
                              jnp.float32)


def _stage_a(x, Wp, Asrc16, Atgt16, Wskip, N):
    R = 1000
    nblk = N // R
    return pl.pallas_call(
        functools.partial(_stage_a_body, nblk),
        grid=(nblk,),
        in_specs=[
            pl.BlockSpec((R, HF), lambda i: (i, 0)),
            pl.BlockSpec((HF, HF), lambda i: (0, 0)),
            pl.BlockSpec((HF, 16), lambda i: (0, 0)),
            pl.BlockSpec((HF, 16), lambda i: (0, 0)),
            pl.BlockSpec((HF, HF), lambda i: (0, 0)),
        ],
        out_specs=[
            pl.BlockSpec((R, HF), lambda i: (i, 0)),
            pl.BlockSpec((R, 16), lambda i: (i, 0)),
            pl.BlockSpec((R, 16), lambda i: (i, 0)),
            pl.BlockSpec((R, HF), lambda i: (i, 0)),
            pl.BlockSpec((8, 128), lambda i: (0, 0)),
        ],
        out_shape=[
            jax.ShapeDtypeStruct((N, HF), jnp.float32),
            jax.ShapeDtypeStruct((N, 16), jnp.float32),
            jax.ShapeDtypeStruct((N, 16), jnp.float32),
            jax.ShapeDtypeStruct((N, HF), jnp.float32),
            jax.ShapeDtypeStruct((8, 128), jnp.float32),
        ],
        scratch_shapes=[pltpu.SMEM((2,), jnp.float32)],
    )(x, Wp, Asrc16, Atgt16, Wskip)


def _edge_pass(proj, ssrc, stgt, esrc, etgt, m16, zerosA, zerosD, N, NP, E):
    EW = E // NW
    NCH = EW // C
    NZR = NP // NS
    mesh = plsc.VectorSubcoreMesh(core_axis_name="c", subcore_axis_name="s",
                                  num_cores=NC, num_subcores=NS)

    @functools.partial(
        pl.kernel,
        out_type=(
            jax.ShapeDtypeStruct((NC, NP, HF), jnp.float32),
            jax.ShapeDtypeStruct((NC, NP, 16), jnp.float32),
        ),
        mesh=mesh,
        scratch_types=[
            pltpu.VMEM_SHARED((NP, HF), jnp.float32),
            pltpu.VMEM_SHARED((NP, 16), jnp.float32),
            pltpu.VMEM((C,), jnp.int32),
            pltpu.VMEM((C,), jnp.int32),
            pltpu.VMEM((C, HF), jnp.float32),
            pltpu.VMEM((C, 16), jnp.float32),
            pltpu.VMEM((C, 16), jnp.float32),
            pltpu.VMEM((C, 16), jnp.float32),
            pltpu.VMEM((C,), jnp.int32),
            pltpu.VMEM((C,), jnp.int32),
            pltpu.VMEM((C, HF), jnp.float32),
            pltpu.VMEM((C, 16), jnp.float32),
            pltpu.VMEM((C, 16), jnp.float32),
            pltpu.VMEM((16,), jnp.float32),
            pltpu.SemaphoreType.DMA,
            pltpu.SemaphoreType.DMA,
        ],
        compiler_params=pltpu.CompilerParams(use_tc_tiling_on_sc=False,
                                             needs_layout_passes=False),
    )
    def k(proj_hbm, ssrc_hbm, stgt_hbm, esrc_hbm, etgt_hbm, m_hbm,
          zerosA_hbm, zerosD_hbm, acc_out, den_out,
          acc_sh, den_sh, src_v, tgt_v, pbuf, wbuf, ssb, stb,
          src_v2, tgt_v2, pbuf2, ssb2, stb2, m_v, semA, semB):
        cid = lax.axis_index("c")
        sid = lax.axis_index("s")
        wid = cid * NS + sid
        r0 = sid * NZR
        pltpu.sync_copy(zerosA_hbm, pbuf)
        pltpu.sync_copy(zerosD_hbm, wbuf)

        def zero_j(j, carry):
            pltpu.sync_copy(pbuf, acc_sh.at[pl.ds(r0 + j * C, C)])
            pltpu.sync_copy(wbuf, den_sh.at[pl.ds(r0 + j * C, C)])
            return carry

        lax.fori_loop(0, NZR // C, zero_j, 0)
        pltpu.sync_copy(m_hbm, m_v)
        m = m_v[...]
        lanes = lax.iota(jnp.int32, 16)
        maskH = lanes < H
        plsc.subcore_barrier()

        bufA = (src_v, tgt_v, pbuf, ssb, stb, semA)
        bufB = (src_v2, tgt_v2, pbuf2, ssb2, stb2, semB)

        def fire(kk, buf):
            sv, tv, pb, sb, tb, sm = buf
            off = (wid * NCH + kk) * C
            pltpu.sync_copy(esrc_hbm.at[pl.ds(off, C)], sv)
            pltpu.sync_copy(etgt_hbm.at[pl.ds(off, C)], tv)
            pltpu.async_copy(proj_hbm.at[sv], pb, sm)
            pltpu.async_copy(ssrc_hbm.at[sv], sb, sm)
            pltpu.async_copy(stgt_hbm.at[tv], tb, sm)

        def drain(buf):
            sv, tv, pb, sb, tb, sm = buf
            pltpu.make_async_copy(proj_hbm.at[sv], pb, sm).wait()
            pltpu.make_async_copy(ssrc_hbm.at[sv], sb, sm).wait()
            pltpu.make_async_copy(stgt_hbm.at[tv], tb, sm).wait()

        def compute_and_scatter(buf):
            sv, tv, pb, sb, tb, sm = buf

            def edge(e, carry2):
                ev = jnp.full((16,), e, jnp.int32)
                srow = plsc.load_gather(sb, [ev, lanes])
                trow = plsc.load_gather(tb, [ev, lanes])
                z = srow + trow
                z = jnp.where(z >= 0.0, z, 0.2 * z)
                w = jnp.where(maskH, jnp.exp(z - m), 0.0)
                plsc.store_scatter(wbuf, [ev, lanes], w)
                for h in range(H):
                    hv = jnp.full((16,), h, jnp.int32)
                    b = plsc.load_gather(wbuf, [ev, hv])
                    for t in range(2):
                        cv = lanes + (2 * h + t) * 16
                        v = plsc.load_gather(pb, [ev, cv]) * b
                        plsc.store_scatter(pb, [ev, cv], v)
                return carry2

            lax.fori_loop(0, C, edge, 0)
            pltpu.sync_copy(pb, acc_sh.at[tv], add=True)
            pltpu.sync_copy(wbuf, den_sh.at[tv], add=True)

        fire(0, bufA)

        def pair(i, carry):
            drain(bufA)
            fire(2 * i + 1, bufB)
            compute_and_scatter(bufA)
            drain(bufB)
            fire(2 * i + 2, bufA)
            compute_and_scatter(bufB)
            return carry

        lax.fori_loop(0, (NCH - 1) // 2, pair, 0)
        drain(bufA)
        compute_and_scatter(bufA)
        plsc.subcore_barrier()

        def dump_j(j, carry):
            rj = r0 + j * C
            pltpu.sync_copy(acc_sh.at[pl.ds(rj, C)], pbuf)
            pltpu.sync_copy(pbuf, acc_out.at[cid, pl.ds(rj, C)])
            pltpu.sync_copy(den_sh.at[pl.ds(rj, C)], wbuf)
            pltpu.sync_copy(wbuf, den_out.at[cid, pl.ds(rj, C)])
            return carry

        lax.fori_loop(0, NZR // C, dump_j, 0)

    return k(proj, ssrc, stgt, esrc, etgt, m16, zerosA, zerosD)


def _stage_c_body(acc_ref, den_ref, skip_ref, g2_ref, out_ref):
    a = acc_ref[0] + acc_ref[1]
    d = den_ref[0] + den_ref[1]
    d128 = lax.dot_general(d, g2_ref[...], (((1,), (0,)), ((), ())),
                           preferred_element_type=jnp.float32,
                           precision=lax.Precision.HIGHEST)
    out_ref[...] = a / (d128 + 1e-16) + skip_ref[...]


def _stage_c(accP, denP, skip, G2, N):
    R = 1000
    nblk = N // R
    return pl.pallas_call(
        _stage_c_body,
        grid=(nblk,),
        in_specs=[
            pl.BlockSpec((NC, R, HF), lambda i: (0, i, 0)),
            pl.BlockSpec((NC, R, 16), lambda i: (0, i, 0)),
            pl.BlockSpec((R, HF), lambda i: (i, 0)),
            pl.BlockSpec((16, HF), lambda i: (0, 0)),
        ],
        out_specs=pl.BlockSpec((R, HF), lambda i: (i, 0)),
        out_shape=jax.ShapeDtypeStruct((N, HF), jnp.float32),
    )(accP, denP, skip, G2)


def kernel(x, edge_index, Wp, a_src, a_tgt, Wskip):
    N = x.shape[0]
    E = edge_index.shape[1]
    eyeH = jnp.eye(H, dtype=jnp.float32)
    A_src = (a_src[0][:, :, None] * eyeH[:, None, :]).reshape(HF, H)
    A_tgt = (a_tgt[0][:, :, None] * eyeH[:, None, :]).reshape(HF, H)
    pad = jnp.zeros((HF, 16 - H), jnp.float32)
    Asrc16 = jnp.concatenate([A_src, pad], axis=1)
    Atgt16 = jnp.concatenate([A_tgt, pad], axis=1)
    G2 = jnp.concatenate(
        [jnp.kron(eyeH, jnp.ones((1, F), jnp.float32)),
         jnp.zeros((16 - H, HF), jnp.float32)], axis=0)
    esrc = edge_index[0].astype(jnp.int32)
    etgt = edge_index[1].astype(jnp.int32)
    NP = ((N + NS * C - 1) // (NS * C)) * NS * C
    zerosA = jnp.zeros((C, HF), jnp.float32)
    zerosD = jnp.zeros((C, 16), jnp.float32)

    proj, ssrc, stgt, skip, m8 = _stage_a(x, Wp, Asrc16, Atgt16, Wskip, N)
    m16 = m8.reshape(-1)[:16]
    accP, denP = _edge_pass(proj, ssrc, stgt, esrc, etgt, m16,
                            zerosA, zerosD, N, NP, E)
    return _stage_c(accP, denP, skip, G2, N)
